# Initial kernel scaffold; baseline (speedup 1.0000x reference)
import jax
import jax.numpy as jnp
from jax.experimental import pallas as pl


def kernel(out1, out2, out3, out4, w1, b1, w2, b2):
    raise NotImplementedError("write your pallas kernel here")


# speedup vs baseline: 2.7758x; 2.7758x over previous
import numpy as np

import jax
import jax.numpy as jnp
from jax.experimental import pallas as pl
from jax.experimental.pallas import tpu as pltpu

_HIGHEST = jax.lax.Precision.HIGHEST


def _interp_matrix_1d(out_size: int, in_size: int) -> np.ndarray:
    if out_size == in_size:
        return np.eye(out_size, dtype=np.float32)
    scale = in_size / out_size
    src = (np.arange(out_size, dtype=np.float64) + 0.5) * scale - 0.5
    src = np.maximum(src, 0.0)
    i0 = np.minimum(np.floor(src).astype(np.int64), in_size - 1)
    i1 = np.minimum(i0 + 1, in_size - 1)
    lam = src - i0
    m = np.zeros((out_size, in_size), dtype=np.float64)
    m[np.arange(out_size), i0] += 1.0 - lam
    m[np.arange(out_size), i1] += lam
    return m.astype(np.float32)


def _asff_kernel(x1_ref, x2_ref, x3_ref, ty2_ref, tx2t_ref, ty3_ref,
                 tx3t_ref, w1t_ref, b1t_ref, w2t_ref, b2t_ref, o_ref):
    C, H, W = o_ref.shape[1], o_ref.shape[2], o_ref.shape[3]
    x1 = x1_ref[0]
    x2 = x2_ref[0]
    x3 = x3_ref[0]
    h2 = x2.shape[1]
    h3 = x3.shape[1]

    def upsample(x, ty, txt, h_in):
        w_in = x.shape[2]
        t = jnp.dot(x.reshape(C * h_in, w_in), txt,
                    preferred_element_type=jnp.float32)
        return jax.lax.dot_general(
            jnp.broadcast_to(ty[None], (C, H, h_in)),
            t.reshape(C, h_in, W),
            (((2,), (1,)), ((0,), (0,))),
            preferred_element_type=jnp.float32)

    up2 = upsample(x2, ty2_ref[...], tx2t_ref[...], h2)
    up3 = upsample(x3, ty3_ref[...], tx3t_ref[...], h3)

    def cmax(v):
        return jnp.max(jnp.max(v, axis=1, keepdims=True),
                       axis=2, keepdims=True).reshape(C, 1)

    g1 = cmax(x1)
    g2 = cmax(up2)
    g3 = cmax(up3)
    gcat = jnp.concatenate([g1, g2, g3, g2], axis=0)

    hid = jnp.maximum(
        jnp.dot(w1t_ref[...], gcat, precision=_HIGHEST,
                preferred_element_type=jnp.float32) + b1t_ref[...], 0.0)
    s = jax.nn.sigmoid(
        jnp.dot(w2t_ref[...], hid, precision=_HIGHEST,
                preferred_element_type=jnp.float32) + b2t_ref[...])

    wa = s[0:C].reshape(C, 1, 1)
    wb = (s[C:2 * C] + s[3 * C:4 * C]).reshape(C, 1, 1)
    wc = s[2 * C:3 * C].reshape(C, 1, 1)
    o_ref[0] = (x1 * wa + up2 * wb + up3 * wc).astype(o_ref.dtype)


def kernel(out1, out2, out3, out4, w1, b1, w2, b2):
    del out4
    B, C, H, W = out1.shape
    h2, w2_ = out2.shape[2], out2.shape[3]
    h3, w3_ = out3.shape[2], out3.shape[3]

    ty2 = jnp.asarray(_interp_matrix_1d(H, h2))
    tx2t = jnp.asarray(_interp_matrix_1d(W, w2_).T)
    ty3 = jnp.asarray(_interp_matrix_1d(H, h3))
    tx3t = jnp.asarray(_interp_matrix_1d(W, w3_).T)

    w1t = w1.T
    b1t = b1[:, None]
    w2t = w2.T
    b2t = b2[:, None]

    return pl.pallas_call(
        _asff_kernel,
        out_shape=jax.ShapeDtypeStruct((B, C, H, W), out1.dtype),
        grid=(B,),
        in_specs=[
            pl.BlockSpec((1, C, H, W), lambda b: (b, 0, 0, 0)),
            pl.BlockSpec((1, C, h2, w2_), lambda b: (b, 0, 0, 0)),
            pl.BlockSpec((1, C, h3, w3_), lambda b: (b, 0, 0, 0)),
            pl.BlockSpec((H, h2), lambda b: (0, 0)),
            pl.BlockSpec((w2_, W), lambda b: (0, 0)),
            pl.BlockSpec((H, h3), lambda b: (0, 0)),
            pl.BlockSpec((w3_, W), lambda b: (0, 0)),
            pl.BlockSpec(w1t.shape, lambda b: (0, 0)),
            pl.BlockSpec(b1t.shape, lambda b: (0, 0)),
            pl.BlockSpec(w2t.shape, lambda b: (0, 0)),
            pl.BlockSpec(b2t.shape, lambda b: (0, 0)),
        ],
        out_specs=pl.BlockSpec((1, C, H, W), lambda b: (b, 0, 0, 0)),
        compiler_params=pltpu.CompilerParams(
            dimension_semantics=("parallel",),
            vmem_limit_bytes=64 * 1024 * 1024),
    )(out1, out2, out3, ty2, tx2t, ty3, tx3t, w1t, b1t, w2t, b2t)
